# Initial kernel scaffold; baseline (speedup 1.0000x reference)
import jax
import jax.numpy as jnp
from jax.experimental import pallas as pl


def kernel(text, token_class_counts, class_counts):
    raise NotImplementedError("write your pallas kernel here")


# speedup vs baseline: 40.8457x; 40.8457x over previous
import functools

import jax
import jax.numpy as jnp
from jax import lax
from jax.experimental import pallas as pl
from jax.experimental.pallas import tpu as pltpu
from jax.experimental.pallas import tpu_sc as plsc

V = 1000
C = 20
VPAD = 1024
CPAD = 32
B = 4096
LSEQ = 200
NTILES = 32
ROWS_PER_TILE = B // NTILES
GROUPS = ROWS_PER_TILE // 16


def _prep_body(tcc_ref, cc_ref, m_ref, bias_ref):
    tcc = tcc_ref[...]
    colsum = jnp.sum(tcc, axis=0, keepdims=True)
    m = jnp.log(tcc) - jnp.log(colsum)
    row = lax.broadcasted_iota(jnp.int32, m.shape, 0)
    m_ref[...] = jnp.where(row == 0, 0.0, m)
    cc = cc_ref[...]
    bias_ref[...] = jnp.where(cc > 0, jnp.log(cc), -jnp.inf)


_prep = pl.pallas_call(
    _prep_body,
    out_shape=(
        jax.ShapeDtypeStruct((VPAD, CPAD), jnp.float32),
        jax.ShapeDtypeStruct((8, CPAD), jnp.float32),
    ),
)


def _sc_mesh():
    return plsc.VectorSubcoreMesh(core_axis_name="c", subcore_axis_name="s")


@functools.partial(
    pl.kernel,
    mesh=_sc_mesh(),
    compiler_params=pltpu.CompilerParams(needs_layout_passes=False),
    out_type=jax.ShapeDtypeStruct((B, CPAD), jnp.float32),
    scratch_types=[
        pltpu.VMEM((CPAD * VPAD,), jnp.float32),
        pltpu.VMEM((LSEQ, ROWS_PER_TILE), jnp.int32),
        pltpu.VMEM((CPAD,), jnp.float32),
        pltpu.VMEM((ROWS_PER_TILE, CPAD), jnp.float32),
    ],
)
def _nb_sc(text_t, table, bias, out, table_v, text_v, bias_v, out_v):
    wid = lax.axis_index("s") * 2 + lax.axis_index("c")
    base = wid * ROWS_PER_TILE
    pltpu.sync_copy(table, table_v)
    pltpu.sync_copy(text_t.at[:, pl.ds(base, ROWS_PER_TILE)], text_v)
    pltpu.sync_copy(bias, bias_v)

    b_lo = bias_v[pl.ds(0, 16)]
    b_hi = bias_v[pl.ds(16, 16)]

    def init_body(r, carry):
        out_v[r, pl.ds(0, 16)] = b_lo
        out_v[r, pl.ds(16, 16)] = b_hi
        return carry

    lax.fori_loop(0, ROWS_PER_TILE, init_body, 0)

    lane = lax.iota(jnp.int32, 16)
    for g in range(GROUPS):
        def l_body(l, accs):
            toks = text_v[l, pl.ds(g * 16, 16)]
            return tuple(
                accs[c] + plsc.load_gather(table_v, [toks + c * VPAD])
                for c in range(C)
            )

        accs = lax.fori_loop(
            0, LSEQ, l_body,
            tuple(jnp.zeros((16,), jnp.float32) for _ in range(C)),
        )
        rows = g * 16 + lane
        for c in range(C):
            plsc.addupdate_scatter(
                out_v, [rows, jnp.full((16,), c, jnp.int32)], accs[c]
            )

    pltpu.sync_copy(out_v, out.at[pl.ds(base, ROWS_PER_TILE), :])


def kernel(text, token_class_counts, class_counts):
    tcc_pad = jnp.zeros((VPAD, CPAD), jnp.float32).at[:V, :C].set(token_class_counts)
    cc_pad = jnp.ones((8, CPAD), jnp.float32).at[:, :C].set(class_counts[None, :])
    m, bias = _prep(tcc_pad, cc_pad)
    table_flat = jnp.reshape(jnp.transpose(m), (CPAD * VPAD,))
    text_t = jnp.transpose(text)
    out = _nb_sc(text_t, table_flat, bias[0])
    return out[:, :C]


# speedup vs baseline: 42.8790x; 1.0498x over previous
import functools

import jax
import jax.numpy as jnp
from jax import lax
from jax.experimental import pallas as pl
from jax.experimental.pallas import tpu as pltpu
from jax.experimental.pallas import tpu_sc as plsc

V = 1000
C = 20
VPAD = 1024
CPAD = 32
B = 4096
LSEQ = 200
NTILES = 32
ROWS_PER_TILE = B // NTILES
GROUPS = ROWS_PER_TILE // 16


def _prep_body(tcc_ref, cc_ref, m_ref, bias_ref):
    tcc = tcc_ref[...]
    colsum = jnp.sum(tcc, axis=0, keepdims=True)
    m = jnp.log(tcc) - jnp.log(colsum)
    row = lax.broadcasted_iota(jnp.int32, m.shape, 0)
    m_ref[...] = jnp.where(row == 0, 0.0, m)
    cc = cc_ref[...]
    bias_ref[...] = jnp.where(cc > 0, jnp.log(cc), -jnp.inf)


_prep = pl.pallas_call(
    _prep_body,
    out_shape=(
        jax.ShapeDtypeStruct((VPAD, CPAD), jnp.float32),
        jax.ShapeDtypeStruct((8, CPAD), jnp.float32),
    ),
)


def _sc_mesh():
    return plsc.VectorSubcoreMesh(core_axis_name="c", subcore_axis_name="s")


@functools.partial(
    pl.kernel,
    mesh=_sc_mesh(),
    compiler_params=pltpu.CompilerParams(needs_layout_passes=False),
    out_type=jax.ShapeDtypeStruct((B, CPAD), jnp.float32),
    scratch_types=[
        pltpu.VMEM((CPAD * VPAD,), jnp.float32),
        pltpu.VMEM((LSEQ, ROWS_PER_TILE), jnp.int32),
        pltpu.VMEM((CPAD,), jnp.float32),
        pltpu.VMEM((ROWS_PER_TILE, CPAD), jnp.float32),
    ],
)
def _nb_sc(text_t, table, bias, out, table_v, text_v, bias_v, out_v):
    wid = lax.axis_index("s") * 2 + lax.axis_index("c")
    base = wid * ROWS_PER_TILE
    pltpu.sync_copy(table, table_v)
    pltpu.sync_copy(text_t.at[:, pl.ds(base, ROWS_PER_TILE)], text_v)
    pltpu.sync_copy(bias, bias_v)

    b_lo = bias_v[pl.ds(0, 16)]
    b_hi = bias_v[pl.ds(16, 16)]

    def init_body(r, carry):
        out_v[r, pl.ds(0, 16)] = b_lo
        out_v[r, pl.ds(16, 16)] = b_hi
        return carry

    lax.fori_loop(0, ROWS_PER_TILE, init_body, 0)

    lane = lax.iota(jnp.int32, 16)
    UNROLL = 4
    for g in range(GROUPS):
        def l_body(i, accs):
            for u in range(UNROLL):
                l = i * UNROLL + u
                toks = text_v[l, pl.ds(g * 16, 16)]
                accs = tuple(
                    accs[c] + plsc.load_gather(table_v, [toks + c * VPAD])
                    for c in range(C)
                )
            return accs

        accs = lax.fori_loop(
            0, LSEQ // UNROLL, l_body,
            tuple(jnp.zeros((16,), jnp.float32) for _ in range(C)),
        )
        rows = g * 16 + lane
        for c in range(C):
            plsc.addupdate_scatter(
                out_v, [rows, jnp.full((16,), c, jnp.int32)], accs[c]
            )

    pltpu.sync_copy(out_v, out.at[pl.ds(base, ROWS_PER_TILE), :])


def kernel(text, token_class_counts, class_counts):
    tcc_pad = jnp.zeros((VPAD, CPAD), jnp.float32).at[:V, :C].set(token_class_counts)
    cc_pad = jnp.ones((8, CPAD), jnp.float32).at[:, :C].set(class_counts[None, :])
    m, bias = _prep(tcc_pad, cc_pad)
    table_flat = jnp.reshape(jnp.transpose(m), (CPAD * VPAD,))
    text_t = jnp.transpose(text)
    out = _nb_sc(text_t, table_flat, bias[0])
    return out[:, :C]


# speedup vs baseline: 56.9878x; 1.3290x over previous
import functools

import jax
import jax.numpy as jnp
from jax import lax
from jax.experimental import pallas as pl
from jax.experimental.pallas import tpu as pltpu
from jax.experimental.pallas import tpu_sc as plsc

V = 1000
C = 20
VPAD = 1024
CPAD = 32
B = 4096
LSEQ = 200
NTILES = 32
ROWS_PER_TILE = B // NTILES
GROUPS = ROWS_PER_TILE // 16


def _prep_body(tcc_ref, cc_ref, m_ref, bias_ref):
    tcc = tcc_ref[...]
    colsum = jnp.sum(tcc, axis=0, keepdims=True)
    m = jnp.log(tcc) - jnp.log(colsum)
    row = lax.broadcasted_iota(jnp.int32, m.shape, 0)
    m_ref[...] = jnp.where(row == 0, 0.0, m)
    cc = cc_ref[...]
    bias_ref[...] = jnp.where(cc > 0, jnp.log(cc), -jnp.inf)


_prep = pl.pallas_call(
    _prep_body,
    out_shape=(
        jax.ShapeDtypeStruct((VPAD, CPAD), jnp.float32),
        jax.ShapeDtypeStruct((8, CPAD), jnp.float32),
    ),
)


def _sc_mesh():
    return plsc.VectorSubcoreMesh(core_axis_name="c", subcore_axis_name="s")


@functools.partial(
    pl.kernel,
    mesh=_sc_mesh(),
    compiler_params=pltpu.CompilerParams(needs_layout_passes=False),
    out_type=jax.ShapeDtypeStruct((B, CPAD), jnp.float32),
    scratch_types=[
        pltpu.VMEM(((C // 2) * VPAD,), jnp.int32),
        pltpu.VMEM((LSEQ, ROWS_PER_TILE), jnp.int32),
        pltpu.VMEM((CPAD,), jnp.float32),
        pltpu.VMEM((ROWS_PER_TILE, CPAD), jnp.float32),
    ],
)
def _nb_sc(text_t, table, bias, out, table_v, text_v, bias_v, out_v):
    wid = lax.axis_index("s") * 2 + lax.axis_index("c")
    base = wid * ROWS_PER_TILE
    pltpu.sync_copy(table, table_v)
    pltpu.sync_copy(text_t.at[:, pl.ds(base, ROWS_PER_TILE)], text_v)
    pltpu.sync_copy(bias, bias_v)

    b_lo = bias_v[pl.ds(0, 16)]
    b_hi = bias_v[pl.ds(16, 16)]

    def init_body(r, carry):
        out_v[r, pl.ds(0, 16)] = b_lo
        out_v[r, pl.ds(16, 16)] = b_hi
        return carry

    lax.fori_loop(0, ROWS_PER_TILE, init_body, 0)

    lane = lax.iota(jnp.int32, 16)
    UNROLL = 4
    for g in range(GROUPS):
        def l_body(i, accs):
            for u in range(UNROLL):
                l = i * UNROLL + u
                toks = text_v[l, pl.ds(g * 16, 16)]
                new = list(accs)
                for w in range(C // 2):
                    word = plsc.load_gather(table_v, [toks + w * VPAD])
                    pair = plsc.bitcast(word, jnp.bfloat16)
                    lo, hi = plsc.unpack(pair, format=plsc.PackFormat.INTERLEAVED)
                    new[2 * w] = new[2 * w] + lo
                    new[2 * w + 1] = new[2 * w + 1] + hi
                accs = tuple(new)
            return accs

        accs = lax.fori_loop(
            0, LSEQ // UNROLL, l_body,
            tuple(jnp.zeros((16,), jnp.float32) for _ in range(C)),
        )
        rows = g * 16 + lane
        for c in range(C):
            plsc.addupdate_scatter(
                out_v, [rows, jnp.full((16,), c, jnp.int32)], accs[c]
            )

    pltpu.sync_copy(out_v, out.at[pl.ds(base, ROWS_PER_TILE), :])


def kernel(text, token_class_counts, class_counts):
    tcc_pad = jnp.zeros((VPAD, CPAD), jnp.float32).at[:V, :C].set(token_class_counts)
    cc_pad = jnp.ones((8, CPAD), jnp.float32).at[:, :C].set(class_counts[None, :])
    m, bias = _prep(tcc_pad, cc_pad)
    mt = jnp.transpose(m)[:C].astype(jnp.bfloat16)
    packed = jax.lax.bitcast_convert_type(
        jnp.swapaxes(jnp.reshape(mt, (C // 2, 2, VPAD)), 1, 2), jnp.int32
    )
    table_flat = jnp.reshape(packed, ((C // 2) * VPAD,))
    text_t = jnp.transpose(text)
    out = _nb_sc(text_t, table_flat, bias[0])
    return out[:, :C]
